# Initial kernel scaffold; baseline (speedup 1.0000x reference)
import jax
import jax.numpy as jnp
from jax.experimental import pallas as pl


def kernel(x, edge_index, batch, Ws0, bs0, Wn0, bn0, Ws1, bs1, Wn1, bn1, Ws2, bs2, Wn2, bn2, Wh1, bh1, Wh2, bh2):
    raise NotImplementedError("write your pallas kernel here")


# speedup vs baseline: 2.9170x; 2.9170x over previous
import functools

import jax
import jax.numpy as jnp
from jax import lax
from jax.experimental import pallas as pl
from jax.experimental.pallas import tpu as pltpu
from jax.experimental.pallas import tpu_sc as plsc

N = 10000
D = 128
G = 16
NC = 2
NS = 16
NW = NC * NS

CHUNK = 128
NP = N + 112
NDP = N + 240
ROWS_PER_TILE = NP // NS
DEG_PER_TILE = NDP // NS

ROW_BLK = 1000


@functools.lru_cache(maxsize=None)
def _make_agg(cpt):
  mesh = plsc.VectorSubcoreMesh(core_axis_name="c", subcore_axis_name="s")

  @functools.partial(
      pl.kernel,
      mesh=mesh,
      out_type=(
          jax.ShapeDtypeStruct((NC, NP, D), jnp.float32),
          jax.ShapeDtypeStruct((NC, NDP), jnp.float32),
      ),
      scratch_types=[
          pltpu.VMEM((cpt, CHUNK), jnp.int32),
          pltpu.VMEM((cpt, CHUNK), jnp.int32),
          pltpu.VMEM((CHUNK, D), jnp.float32),
          pltpu.VMEM((CHUNK,), jnp.float32),
          pltpu.VMEM_SHARED((NP, D), jnp.float32),
          pltpu.VMEM_SHARED((NDP,), jnp.float32),
          pltpu.SemaphoreType.DMA,
      ],
  )
  def agg(h, src2, dst2, zrows, zflat, ones_in, pacc, pdeg,
          src_v, dst_v, rows_v, ones_v, acc_sh, deg_sh, sem):
    cid = lax.axis_index("c")
    sid = lax.axis_index("s")
    wid = cid * NS + sid
    r0 = pl.multiple_of(sid * ROWS_PER_TILE, 8)
    d0 = pl.multiple_of(sid * DEG_PER_TILE, 8)
    c0 = pl.multiple_of(wid * cpt, 8)

    pltpu.sync_copy(zrows.at[pl.ds(r0, ROWS_PER_TILE), :],
                    acc_sh.at[pl.ds(r0, ROWS_PER_TILE), :])
    pltpu.sync_copy(zflat.at[pl.ds(d0, DEG_PER_TILE)],
                    deg_sh.at[pl.ds(d0, DEG_PER_TILE)])
    pltpu.sync_copy(ones_in, ones_v)
    pltpu.sync_copy(src2.at[pl.ds(c0, cpt), :], src_v)
    pltpu.sync_copy(dst2.at[pl.ds(c0, cpt), :], dst_v)
    plsc.subcore_barrier()

    def chunk(j, carry):
      pltpu.async_copy(h.at[src_v.at[j]], rows_v, sem).wait()
      pltpu.sync_copy(rows_v, acc_sh.at[dst_v.at[j]], add=True)
      pltpu.sync_copy(ones_v, deg_sh.at[dst_v.at[j]], add=True)
      return carry

    lax.fori_loop(0, cpt, chunk, 0)
    plsc.subcore_barrier()

    pltpu.sync_copy(acc_sh.at[pl.ds(r0, ROWS_PER_TILE), :],
                    pacc.at[cid, pl.ds(r0, ROWS_PER_TILE), :])
    pltpu.sync_copy(deg_sh.at[pl.ds(d0, DEG_PER_TILE)],
                    pdeg.at[cid, pl.ds(d0, DEG_PER_TILE)])

  return agg


def _layer_body(h_ref, p_ref, deg_ref, ws_ref, wn_ref, b_ref, o_ref):
  dsum = deg_ref[0] + deg_ref[1]
  agg = (p_ref[0] + p_ref[1]) / jnp.maximum(dsum, 1.0)
  hs = lax.dot_general(h_ref[...], ws_ref[...], (((1,), (1,)), ((), ())),
                       preferred_element_type=jnp.float32)
  hn = lax.dot_general(agg, wn_ref[...], (((1,), (1,)), ((), ())),
                       preferred_element_type=jnp.float32)
  o_ref[...] = jnp.maximum(hs + hn + b_ref[...], 0.0)


def _layer_tc(h, pacc, deg3, ws, wn, b):
  nblk = N // ROW_BLK
  return pl.pallas_call(
      _layer_body,
      grid=(nblk,),
      in_specs=[
          pl.BlockSpec((ROW_BLK, D), lambda i: (i, 0)),
          pl.BlockSpec((NC, ROW_BLK, D), lambda i: (0, i, 0)),
          pl.BlockSpec((NC, ROW_BLK, 1), lambda i: (0, i, 0)),
          pl.BlockSpec((D, D), lambda i: (0, 0)),
          pl.BlockSpec((D, D), lambda i: (0, 0)),
          pl.BlockSpec((1, D), lambda i: (0, 0)),
      ],
      out_specs=pl.BlockSpec((ROW_BLK, D), lambda i: (i, 0)),
      out_shape=jax.ShapeDtypeStruct((N, D), jnp.float32),
  )(h, pacc, deg3, ws, wn, b)


def _pool_body(h_ref, bt_ref, w1_ref, b1_ref, w2_ref, o_ref,
               gsum, cnt):
  i = pl.program_id(0)

  @pl.when(i == 0)
  def _():
    gsum[...] = jnp.zeros_like(gsum)
    cnt[...] = jnp.zeros_like(cnt)

  oh = (bt_ref[...] == lax.broadcasted_iota(jnp.int32, (ROW_BLK, G), 1))
  oh = oh.astype(jnp.float32)
  gsum[...] += lax.dot_general(oh, h_ref[...], (((0,), (0,)), ((), ())),
                               preferred_element_type=jnp.float32)
  cnt[...] += jnp.sum(oh, axis=0)[:, None]

  @pl.when(i == pl.num_programs(0) - 1)
  def _():
    g = gsum[...] / jnp.maximum(cnt[...], 1.0)
    hh = lax.dot_general(g, w1_ref[...], (((1,), (1,)), ((), ())),
                         preferred_element_type=jnp.float32) + b1_ref[...]
    hh = jnp.maximum(hh, 0.0)
    o_ref[...] = jnp.sum(hh * w2_ref[...], axis=1, keepdims=True)


def _pool_tc(h, batch2, wh1, bh1, wh2):
  nblk = N // ROW_BLK
  return pl.pallas_call(
      _pool_body,
      grid=(nblk,),
      in_specs=[
          pl.BlockSpec((ROW_BLK, D), lambda i: (i, 0)),
          pl.BlockSpec((ROW_BLK, 1), lambda i: (i, 0)),
          pl.BlockSpec((D, D), lambda i: (0, 0)),
          pl.BlockSpec((1, D), lambda i: (0, 0)),
          pl.BlockSpec((1, D), lambda i: (0, 0)),
      ],
      out_specs=pl.BlockSpec((G, 1), lambda i: (0, 0)),
      out_shape=jax.ShapeDtypeStruct((G, 1), jnp.float32),
      scratch_shapes=[
          pltpu.VMEM((G, D), jnp.float32),
          pltpu.VMEM((G, 1), jnp.float32),
      ],
  )(h, batch2, wh1, bh1, wh2)


def kernel(x, edge_index, batch, Ws0, bs0, Wn0, bn0, Ws1, bs1, Wn1, bn1,
           Ws2, bs2, Wn2, bn2, Wh1, bh1, Wh2, bh2):
  e = edge_index.shape[1]
  cpt = -(-e // (NW * CHUNK))
  cpt = -(-cpt // 8) * 8
  epad = NW * cpt * CHUNK
  pad = epad - e
  src = jnp.concatenate([edge_index[0],
                         jnp.zeros((pad,), jnp.int32)]).reshape(-1, CHUNK)
  dst = jnp.concatenate([edge_index[1],
                         jnp.full((pad,), N, jnp.int32)]).reshape(-1, CHUNK)
  zrows = jnp.zeros((NP, D), jnp.float32)
  zflat = jnp.zeros((NDP,), jnp.float32)
  ones_in = jnp.ones((CHUNK,), jnp.float32)
  agg = _make_agg(cpt)

  h = x
  for ws, bs, wn, bn in ((Ws0, bs0, Wn0, bn0), (Ws1, bs1, Wn1, bn1),
                         (Ws2, bs2, Wn2, bn2)):
    pacc, pdeg = agg(h, src, dst, zrows, zflat, ones_in)
    h = _layer_tc(h, pacc, pdeg.reshape(NC, NDP, 1), ws, wn,
                  (bs + bn).reshape(1, D))

  out = _pool_tc(h, batch.reshape(N, 1), Wh1, bh1.reshape(1, D), Wh2)
  return out.reshape(-1) + bh2


# speedup vs baseline: 4.2210x; 1.4470x over previous
import functools

import jax
import jax.numpy as jnp
from jax import lax
from jax.experimental import pallas as pl
from jax.experimental.pallas import tpu as pltpu
from jax.experimental.pallas import tpu_sc as plsc

N = 10000
D = 128
HD = D // 2
G = 16
NC = 2
NS = 16
NW = NC * NS

CHUNK = 128
RND = 4
SUP = 8
NP = N + 112
NDP = N + 240
ROWS_PER_TILE = NP // NS
DEG_PER_TILE = NDP // NS

ROW_BLK = 1000


@functools.lru_cache(maxsize=None)
def _make_agg(cpt, with_deg):
  mesh = plsc.VectorSubcoreMesh(core_axis_name="c", subcore_axis_name="s")
  nsp = cpt // (2 * SUP)

  out_type = [jax.ShapeDtypeStruct((NC, NP, HD), jnp.float32)]
  if with_deg:
    out_type.append(jax.ShapeDtypeStruct((NC, NDP), jnp.float32))

  @functools.partial(
      pl.kernel,
      mesh=mesh,
      compiler_params=pltpu.CompilerParams(use_tc_tiling_on_sc=False),
      out_type=tuple(out_type),
      scratch_types=[
          pltpu.VMEM((2, SUP, CHUNK), jnp.int32),
          pltpu.VMEM((2, SUP, CHUNK), jnp.int32),
          pltpu.VMEM((2 * RND, CHUNK, HD), jnp.float32),
          pltpu.VMEM((CHUNK,), jnp.float32),
          pltpu.VMEM_SHARED((NP, HD), jnp.float32),
          pltpu.VMEM_SHARED((NDP,), jnp.float32),
          pltpu.SemaphoreType.DMA,
          pltpu.SemaphoreType.DMA,
          pltpu.SemaphoreType.DMA,
          pltpu.SemaphoreType.DMA,
      ],
  )
  def agg(h2, srcs, dst2, zrows, zflat, ones_in, *rest):
    if with_deg:
      pacc, pdeg = rest[0], rest[1]
      rest = rest[2:]
    else:
      pacc = rest[0]
      rest = rest[1:]
    sidx, didx, rows_v, ones_v, acc_sh, deg_sh, sem_g, sem_i, ss0, ss1 = rest
    sem_s = (ss0, ss1)

    cid = lax.axis_index("c")
    sid = lax.axis_index("s")
    tb = pl.multiple_of(sid * cpt, 8)
    r0 = pl.multiple_of(sid * ROWS_PER_TILE, 8)
    d0 = pl.multiple_of(sid * DEG_PER_TILE, 8)

    def gather_copies(slot, rowbase, grp, make):
      mk = pltpu.make_async_copy if make else pltpu.async_copy
      return [mk(h2.at[sidx.at[slot, rowbase + b]], rows_v.at[grp * RND + b],
                 sem_g) for b in range(RND)]

    def issue_gathers(slot, rowbase, grp):
      gather_copies(slot, rowbase, grp, False)

    def drain_gathers(slot, rowbase, grp):
      for c in gather_copies(slot, rowbase, grp, True):
        c.wait()

    def scatter_copies(slot, rowbase, grp, make):
      out = []
      for b in range(RND):
        args_r = (rows_v.at[grp * RND + b],
                  acc_sh.at[didx.at[slot, rowbase + b]], sem_s[grp])
        args_d = (ones_v, deg_sh.at[didx.at[slot, rowbase + b]], sem_s[grp])
        if make:
          out.append(pltpu.make_async_copy(*args_r))
          if with_deg:
            out.append(pltpu.make_async_copy(*args_d))
        else:
          out.append(pltpu.async_copy(*args_r, add=True))
          if with_deg:
            out.append(pltpu.async_copy(*args_d, add=True))
      return out

    def issue_scatters(slot, rowbase, grp):
      scatter_copies(slot, rowbase, grp, False)

    def drain_scatters(slot, rowbase, grp):
      for c in scatter_copies(slot, rowbase, grp, True):
        c.wait()

    def prefetch_idx(s, slot):
      off = pl.multiple_of(tb + s * SUP, 8)
      pltpu.async_copy(srcs.at[cid, pl.ds(off, SUP), :], sidx.at[slot], sem_i)
      pltpu.async_copy(dst2.at[pl.ds(off, SUP), :], didx.at[slot], sem_i)

    def wait_idx(slot):
      pltpu.make_async_copy(srcs.at[cid, pl.ds(tb, SUP), :], sidx.at[slot],
                            sem_i).wait()
      pltpu.make_async_copy(dst2.at[pl.ds(tb, SUP), :], didx.at[slot],
                            sem_i).wait()

    pltpu.sync_copy(zrows.at[pl.ds(r0, ROWS_PER_TILE), :],
                    acc_sh.at[pl.ds(r0, ROWS_PER_TILE), :])
    if with_deg:
      pltpu.sync_copy(zflat.at[pl.ds(d0, DEG_PER_TILE)],
                      deg_sh.at[pl.ds(d0, DEG_PER_TILE)])
    pltpu.sync_copy(ones_in, ones_v)
    pltpu.sync_copy(srcs.at[cid, pl.ds(tb, SUP), :], sidx.at[0])
    pltpu.sync_copy(dst2.at[pl.ds(tb, SUP), :], didx.at[0])
    plsc.subcore_barrier()
    issue_gathers(0, 0, 0)

    def body(sp, carry):
      drain_gathers(0, 0, 0)
      issue_scatters(0, 0, 0)

      @pl.when(sp > 0)
      def _():
        drain_scatters(1, RND, 1)
      prefetch_idx(2 * sp + 1, 1)
      issue_gathers(0, RND, 1)

      drain_gathers(0, RND, 1)
      issue_scatters(0, RND, 1)
      drain_scatters(0, 0, 0)
      wait_idx(1)
      issue_gathers(1, 0, 0)

      drain_gathers(1, 0, 0)
      issue_scatters(1, 0, 0)
      drain_scatters(0, RND, 1)

      @pl.when(sp < nsp - 1)
      def _():
        prefetch_idx(2 * sp + 2, 0)
      issue_gathers(1, RND, 1)

      drain_gathers(1, RND, 1)
      issue_scatters(1, RND, 1)
      drain_scatters(1, 0, 0)

      @pl.when(sp < nsp - 1)
      def _():
        wait_idx(0)
        issue_gathers(0, 0, 0)
      return carry

    lax.fori_loop(0, nsp, body, 0)
    drain_scatters(1, RND, 1)
    plsc.subcore_barrier()

    pltpu.sync_copy(acc_sh.at[pl.ds(r0, ROWS_PER_TILE), :],
                    pacc.at[cid, pl.ds(r0, ROWS_PER_TILE), :])
    if with_deg:
      pltpu.sync_copy(deg_sh.at[pl.ds(d0, DEG_PER_TILE)],
                      pdeg.at[cid, pl.ds(d0, DEG_PER_TILE)])

  return agg


def _layer_body(h_ref, p_ref, deg_ref, ws_ref, wn_ref, b_ref, o_ref):
  agg = jnp.concatenate([p_ref[0], p_ref[1]], axis=1)
  agg = agg / jnp.maximum(deg_ref[0], 1.0)
  hs = lax.dot_general(h_ref[...], ws_ref[...], (((1,), (1,)), ((), ())),
                       preferred_element_type=jnp.float32)
  hn = lax.dot_general(agg, wn_ref[...], (((1,), (1,)), ((), ())),
                       preferred_element_type=jnp.float32)
  o_ref[...] = jnp.maximum(hs + hn + b_ref[...], 0.0)


def _layer_tc(h, pacc, deg3, ws, wn, b):
  nblk = N // ROW_BLK
  return pl.pallas_call(
      _layer_body,
      grid=(nblk,),
      in_specs=[
          pl.BlockSpec((ROW_BLK, D), lambda i: (i, 0)),
          pl.BlockSpec((NC, ROW_BLK, HD), lambda i: (0, i, 0)),
          pl.BlockSpec((1, ROW_BLK, 1), lambda i: (0, i, 0)),
          pl.BlockSpec((D, D), lambda i: (0, 0)),
          pl.BlockSpec((D, D), lambda i: (0, 0)),
          pl.BlockSpec((1, D), lambda i: (0, 0)),
      ],
      out_specs=pl.BlockSpec((ROW_BLK, D), lambda i: (i, 0)),
      out_shape=jax.ShapeDtypeStruct((N, D), jnp.float32),
  )(h, pacc, deg3, ws, wn, b)


def _pool_body(h_ref, bt_ref, w1_ref, b1_ref, w2_ref, o_ref,
               gsum, cnt):
  i = pl.program_id(0)

  @pl.when(i == 0)
  def _():
    gsum[...] = jnp.zeros_like(gsum)
    cnt[...] = jnp.zeros_like(cnt)

  oh = (bt_ref[...] == lax.broadcasted_iota(jnp.int32, (ROW_BLK, G), 1))
  oh = oh.astype(jnp.float32)
  gsum[...] += lax.dot_general(oh, h_ref[...], (((0,), (0,)), ((), ())),
                               preferred_element_type=jnp.float32)
  cnt[...] += jnp.sum(oh, axis=0)[:, None]

  @pl.when(i == pl.num_programs(0) - 1)
  def _():
    g = gsum[...] / jnp.maximum(cnt[...], 1.0)
    hh = lax.dot_general(g, w1_ref[...], (((1,), (1,)), ((), ())),
                         preferred_element_type=jnp.float32) + b1_ref[...]
    hh = jnp.maximum(hh, 0.0)
    o_ref[...] = jnp.sum(hh * w2_ref[...], axis=1, keepdims=True)


def _pool_tc(h, batch2, wh1, bh1, wh2):
  nblk = N // ROW_BLK
  return pl.pallas_call(
      _pool_body,
      grid=(nblk,),
      in_specs=[
          pl.BlockSpec((ROW_BLK, D), lambda i: (i, 0)),
          pl.BlockSpec((ROW_BLK, 1), lambda i: (i, 0)),
          pl.BlockSpec((D, D), lambda i: (0, 0)),
          pl.BlockSpec((1, D), lambda i: (0, 0)),
          pl.BlockSpec((1, D), lambda i: (0, 0)),
      ],
      out_specs=pl.BlockSpec((G, 1), lambda i: (0, 0)),
      out_shape=jax.ShapeDtypeStruct((G, 1), jnp.float32),
      scratch_shapes=[
          pltpu.VMEM((G, D), jnp.float32),
          pltpu.VMEM((G, 1), jnp.float32),
      ],
  )(h, batch2, wh1, bh1, wh2)


def kernel(x, edge_index, batch, Ws0, bs0, Wn0, bn0, Ws1, bs1, Wn1, bn1,
           Ws2, bs2, Wn2, bn2, Wh1, bh1, Wh2, bh2):
  e = edge_index.shape[1]
  cpt = -(-e // (NS * CHUNK))
  cpt = -(-cpt // (2 * SUP)) * (2 * SUP)
  epad = NS * cpt * CHUNK
  pad = epad - e
  src = jnp.concatenate([edge_index[0], jnp.zeros((pad,), jnp.int32)])
  srcs = jnp.stack([src * 2, src * 2 + 1]).reshape(NC, -1, CHUNK)
  dst = jnp.concatenate([edge_index[1],
                         jnp.full((pad,), N, jnp.int32)]).reshape(-1, CHUNK)
  zrows = jnp.zeros((NP, HD), jnp.float32)
  zflat = jnp.zeros((NDP,), jnp.float32)
  ones_in = jnp.ones((CHUNK,), jnp.float32)
  agg_first = _make_agg(cpt, True)
  agg_rest = _make_agg(cpt, False)

  h = x
  pdeg = None
  for li, (ws, bs, wn, bn) in enumerate(
      ((Ws0, bs0, Wn0, bn0), (Ws1, bs1, Wn1, bn1), (Ws2, bs2, Wn2, bn2))):
    h2 = h.reshape(N * 2, HD)
    if li == 0:
      pacc, pdeg = agg_first(h2, srcs, dst, zrows, zflat, ones_in)
    else:
      (pacc,) = agg_rest(h2, srcs, dst, zrows, zflat, ones_in)
    h = _layer_tc(h, pacc, pdeg.reshape(NC, NDP, 1), ws, wn,
                  (bs + bn).reshape(1, D))

  out = _pool_tc(h, batch.reshape(N, 1), Wh1, bh1.reshape(1, D), Wh2)
  return out.reshape(-1) + bh2


# speedup vs baseline: 8.6297x; 2.0445x over previous
import functools

import jax
import jax.numpy as jnp
from jax import lax
from jax.experimental import pallas as pl
from jax.experimental.pallas import tpu as pltpu
from jax.experimental.pallas import tpu_sc as plsc

N = 10000
D = 128
HD = D // 2
G = 16
NC = 2
NS = 16
NW = NC * NS

CHUNK = 128
RND = 2
SUP = 8
NP = N + 112
NDP = N + 240
ROWS_PER_TILE = NP // NS
DEG_PER_TILE = NDP // NS

ROW_BLK = 1000


@functools.lru_cache(maxsize=None)
def _make_agg(cpt, with_deg):
  mesh = plsc.VectorSubcoreMesh(core_axis_name="c", subcore_axis_name="s")
  nsp = cpt // (2 * SUP)

  out_type = [jax.ShapeDtypeStruct((NC, NP, HD), jnp.float32)]
  if with_deg:
    out_type.append(jax.ShapeDtypeStruct((NC, NDP), jnp.float32))

  @functools.partial(
      pl.kernel,
      mesh=mesh,
      compiler_params=pltpu.CompilerParams(use_tc_tiling_on_sc=False),
      out_type=tuple(out_type),
      scratch_types=[
          pltpu.VMEM((2, SUP, CHUNK), jnp.int32),
          pltpu.VMEM((2, SUP, CHUNK), jnp.int32),
          pltpu.VMEM((2 * RND, CHUNK, HD), jnp.float32),
          pltpu.VMEM((CHUNK,), jnp.float32),
          pltpu.VMEM_SHARED((NP, HD), jnp.float32),
          pltpu.VMEM_SHARED((NP, HD), jnp.float32),
          pltpu.VMEM_SHARED((NDP,), jnp.float32),
          pltpu.SemaphoreType.DMA,
          pltpu.SemaphoreType.DMA,
          pltpu.SemaphoreType.DMA,
          pltpu.SemaphoreType.DMA,
      ],
  )
  def agg(h_pad, src2, dst2, zrows, zflat, ones_in, *rest):
    if with_deg:
      pacc, pdeg = rest[0], rest[1]
      rest = rest[2:]
    else:
      pacc = rest[0]
      rest = rest[1:]
    (sidx, didx, rows_v, ones_v, h_sh, acc_sh, deg_sh,
     sem_g, sem_i, ss0, ss1) = rest
    sem_s = (ss0, ss1)

    cid = lax.axis_index("c")
    sid = lax.axis_index("s")
    tb = pl.multiple_of(sid * cpt, 8)
    r0 = pl.multiple_of(sid * ROWS_PER_TILE, 8)
    d0 = pl.multiple_of(sid * DEG_PER_TILE, 8)

    def gather_copies(slot, rowbase, grp, make):
      mk = pltpu.make_async_copy if make else pltpu.async_copy
      return [mk(h_sh.at[sidx.at[slot, rowbase + b]], rows_v.at[grp * RND + b],
                 sem_g) for b in range(RND)]

    def issue_gathers(slot, rowbase, grp):
      gather_copies(slot, rowbase, grp, False)

    def drain_gathers(slot, rowbase, grp):
      for c in gather_copies(slot, rowbase, grp, True):
        c.wait()

    def scatter_copies(slot, rowbase, grp, make):
      out = []
      for b in range(RND):
        args_r = (rows_v.at[grp * RND + b],
                  acc_sh.at[didx.at[slot, rowbase + b]], sem_s[grp])
        args_d = (ones_v, deg_sh.at[didx.at[slot, rowbase + b]], sem_s[grp])
        if make:
          out.append(pltpu.make_async_copy(*args_r))
          if with_deg:
            out.append(pltpu.make_async_copy(*args_d))
        else:
          out.append(pltpu.async_copy(*args_r, add=True))
          if with_deg:
            out.append(pltpu.async_copy(*args_d, add=True))
      return out

    def issue_scatters(slot, rowbase, grp):
      scatter_copies(slot, rowbase, grp, False)

    def drain_scatters(slot, rowbase, grp):
      for c in scatter_copies(slot, rowbase, grp, True):
        c.wait()

    def prefetch_idx(s, slot):
      off = pl.multiple_of(tb + s * SUP, 8)
      pltpu.async_copy(src2.at[pl.ds(off, SUP), :], sidx.at[slot], sem_i)
      pltpu.async_copy(dst2.at[pl.ds(off, SUP), :], didx.at[slot], sem_i)

    def wait_idx(slot):
      pltpu.make_async_copy(src2.at[pl.ds(tb, SUP), :], sidx.at[slot],
                            sem_i).wait()
      pltpu.make_async_copy(dst2.at[pl.ds(tb, SUP), :], didx.at[slot],
                            sem_i).wait()

    pltpu.sync_copy(h_pad.at[cid, pl.ds(r0, ROWS_PER_TILE), :],
                    h_sh.at[pl.ds(r0, ROWS_PER_TILE), :])
    pltpu.sync_copy(zrows.at[pl.ds(r0, ROWS_PER_TILE), :],
                    acc_sh.at[pl.ds(r0, ROWS_PER_TILE), :])
    if with_deg:
      pltpu.sync_copy(zflat.at[pl.ds(d0, DEG_PER_TILE)],
                      deg_sh.at[pl.ds(d0, DEG_PER_TILE)])
    pltpu.sync_copy(ones_in, ones_v)
    pltpu.sync_copy(src2.at[pl.ds(tb, SUP), :], sidx.at[0])
    pltpu.sync_copy(dst2.at[pl.ds(tb, SUP), :], didx.at[0])
    plsc.subcore_barrier()
    issue_gathers(0, 0, 0)

    nr = 2 * SUP // RND
    rps = SUP // RND
    pos = [((k * RND) // SUP, (k * RND) % SUP, k % 2) for k in range(nr)]

    def body(sp, carry):
      for k in range(nr):
        slot, rowbase, grp = pos[k]
        drain_gathers(slot, rowbase, grp)
        issue_scatters(slot, rowbase, grp)
        pslot, prowbase, pgrp = pos[k - 1]
        if k == 0:
          @pl.when(sp > 0)
          def _():
            drain_scatters(pslot, prowbase, pgrp)
          prefetch_idx(2 * sp + 1, 1)
        else:
          drain_scatters(pslot, prowbase, pgrp)
        if k == rps:
          @pl.when(sp < nsp - 1)
          def _():
            prefetch_idx(2 * sp + 2, 0)
        if k + 1 < nr:
          nslot, nrowbase, ngrp = pos[k + 1]
          if nslot != slot:
            wait_idx(nslot)
          issue_gathers(nslot, nrowbase, ngrp)
        else:
          @pl.when(sp < nsp - 1)
          def _():
            wait_idx(0)
            issue_gathers(0, 0, 0)
      return carry

    lax.fori_loop(0, nsp, body, 0)
    drain_scatters(*pos[nr - 1])
    plsc.subcore_barrier()

    pltpu.sync_copy(acc_sh.at[pl.ds(r0, ROWS_PER_TILE), :],
                    pacc.at[cid, pl.ds(r0, ROWS_PER_TILE), :])
    if with_deg:
      pltpu.sync_copy(deg_sh.at[pl.ds(d0, DEG_PER_TILE)],
                      pdeg.at[cid, pl.ds(d0, DEG_PER_TILE)])

  return agg


def _layer_body(h_ref, p_ref, deg_ref, ws_ref, wn_ref, b_ref, o_ref):
  agg = jnp.concatenate([p_ref[0], p_ref[1]], axis=1)
  agg = agg / jnp.maximum(deg_ref[0], 1.0)
  hs = lax.dot_general(h_ref[...], ws_ref[...], (((1,), (1,)), ((), ())),
                       preferred_element_type=jnp.float32)
  hn = lax.dot_general(agg, wn_ref[...], (((1,), (1,)), ((), ())),
                       preferred_element_type=jnp.float32)
  o_ref[...] = jnp.maximum(hs + hn + b_ref[...], 0.0)


def _layer_tc(h, pacc, deg3, ws, wn, b):
  nblk = N // ROW_BLK
  return pl.pallas_call(
      _layer_body,
      grid=(nblk,),
      in_specs=[
          pl.BlockSpec((ROW_BLK, D), lambda i: (i, 0)),
          pl.BlockSpec((NC, ROW_BLK, HD), lambda i: (0, i, 0)),
          pl.BlockSpec((1, ROW_BLK, 1), lambda i: (0, i, 0)),
          pl.BlockSpec((D, D), lambda i: (0, 0)),
          pl.BlockSpec((D, D), lambda i: (0, 0)),
          pl.BlockSpec((1, D), lambda i: (0, 0)),
      ],
      out_specs=pl.BlockSpec((ROW_BLK, D), lambda i: (i, 0)),
      out_shape=jax.ShapeDtypeStruct((N, D), jnp.float32),
  )(h, pacc, deg3, ws, wn, b)


def _pool_body(h_ref, bt_ref, w1_ref, b1_ref, w2_ref, o_ref,
               gsum, cnt):
  i = pl.program_id(0)

  @pl.when(i == 0)
  def _():
    gsum[...] = jnp.zeros_like(gsum)
    cnt[...] = jnp.zeros_like(cnt)

  oh = (bt_ref[...] == lax.broadcasted_iota(jnp.int32, (ROW_BLK, G), 1))
  oh = oh.astype(jnp.float32)
  gsum[...] += lax.dot_general(oh, h_ref[...], (((0,), (0,)), ((), ())),
                               preferred_element_type=jnp.float32)
  cnt[...] += jnp.sum(oh, axis=0)[:, None]

  @pl.when(i == pl.num_programs(0) - 1)
  def _():
    g = gsum[...] / jnp.maximum(cnt[...], 1.0)
    hh = lax.dot_general(g, w1_ref[...], (((1,), (1,)), ((), ())),
                         preferred_element_type=jnp.float32) + b1_ref[...]
    hh = jnp.maximum(hh, 0.0)
    o_ref[...] = jnp.sum(hh * w2_ref[...], axis=1, keepdims=True)


def _pool_tc(h, batch2, wh1, bh1, wh2):
  nblk = N // ROW_BLK
  return pl.pallas_call(
      _pool_body,
      grid=(nblk,),
      in_specs=[
          pl.BlockSpec((ROW_BLK, D), lambda i: (i, 0)),
          pl.BlockSpec((ROW_BLK, 1), lambda i: (i, 0)),
          pl.BlockSpec((D, D), lambda i: (0, 0)),
          pl.BlockSpec((1, D), lambda i: (0, 0)),
          pl.BlockSpec((1, D), lambda i: (0, 0)),
      ],
      out_specs=pl.BlockSpec((G, 1), lambda i: (0, 0)),
      out_shape=jax.ShapeDtypeStruct((G, 1), jnp.float32),
      scratch_shapes=[
          pltpu.VMEM((G, D), jnp.float32),
          pltpu.VMEM((G, 1), jnp.float32),
      ],
  )(h, batch2, wh1, bh1, wh2)


def kernel(x, edge_index, batch, Ws0, bs0, Wn0, bn0, Ws1, bs1, Wn1, bn1,
           Ws2, bs2, Wn2, bn2, Wh1, bh1, Wh2, bh2):
  e = edge_index.shape[1]
  cpt = -(-e // (NS * CHUNK))
  cpt = -(-cpt // (2 * SUP)) * (2 * SUP)
  epad = NS * cpt * CHUNK
  pad = epad - e
  src = jnp.concatenate([edge_index[0],
                         jnp.zeros((pad,), jnp.int32)]).reshape(-1, CHUNK)
  dst = jnp.concatenate([edge_index[1],
                         jnp.full((pad,), N, jnp.int32)]).reshape(-1, CHUNK)
  rowpad = jnp.zeros((NC, NP - N, HD), jnp.float32)
  zrows = jnp.zeros((NP, HD), jnp.float32)
  zflat = jnp.zeros((NDP,), jnp.float32)
  ones_in = jnp.ones((CHUNK,), jnp.float32)
  agg_first = _make_agg(cpt, True)
  agg_rest = _make_agg(cpt, False)

  h = x
  pdeg = None
  for li, (ws, bs, wn, bn) in enumerate(
      ((Ws0, bs0, Wn0, bn0), (Ws1, bs1, Wn1, bn1), (Ws2, bs2, Wn2, bn2))):
    h_pad = jnp.concatenate(
        [h.reshape(N, NC, HD).transpose(1, 0, 2), rowpad], axis=1)
    if li == 0:
      pacc, pdeg = agg_first(h_pad, src, dst, zrows, zflat, ones_in)
    else:
      (pacc,) = agg_rest(h_pad, src, dst, zrows, zflat, ones_in)
    h = _layer_tc(h, pacc, pdeg.reshape(NC, NDP, 1), ws, wn,
                  (bs + bn).reshape(1, D))

  out = _pool_tc(h, batch.reshape(N, 1), Wh1, bh1.reshape(1, D), Wh2)
  return out.reshape(-1) + bh2


# speedup vs baseline: 8.6379x; 1.0009x over previous
import functools

import jax
import jax.numpy as jnp
from jax import lax
from jax.experimental import pallas as pl
from jax.experimental.pallas import tpu as pltpu
from jax.experimental.pallas import tpu_sc as plsc

N = 10000
D = 128
HD = D // 2
G = 16
NC = 2
NS = 16
NW = NC * NS

CHUNK = 64
RND = 4
SUP = 8
NP = N + 112
NDP = N + 240
ROWS_PER_TILE = NP // NS
DEG_PER_TILE = NDP // NS

ROW_BLK = 1000


@functools.lru_cache(maxsize=None)
def _make_agg(cpt, with_deg):
  mesh = plsc.VectorSubcoreMesh(core_axis_name="c", subcore_axis_name="s")
  nsp = cpt // (2 * SUP)

  out_type = [jax.ShapeDtypeStruct((NC, NP, HD), jnp.float32)]
  if with_deg:
    out_type.append(jax.ShapeDtypeStruct((NC, NDP), jnp.float32))

  @functools.partial(
      pl.kernel,
      mesh=mesh,
      compiler_params=pltpu.CompilerParams(use_tc_tiling_on_sc=False),
      out_type=tuple(out_type),
      scratch_types=[
          pltpu.VMEM((2, SUP, CHUNK), jnp.int32),
          pltpu.VMEM((2, SUP, CHUNK), jnp.int32),
          pltpu.VMEM((2 * RND, CHUNK, HD), jnp.float32),
          pltpu.VMEM((CHUNK,), jnp.float32),
          pltpu.VMEM_SHARED((NP, HD), jnp.float32),
          pltpu.VMEM_SHARED((NP, HD), jnp.float32),
          pltpu.VMEM_SHARED((NDP,), jnp.float32),
          pltpu.SemaphoreType.DMA,
          pltpu.SemaphoreType.DMA,
          pltpu.SemaphoreType.DMA,
          pltpu.SemaphoreType.DMA,
      ],
  )
  def agg(h_pad, src2, dst2, zrows, zflat, ones_in, *rest):
    if with_deg:
      pacc, pdeg = rest[0], rest[1]
      rest = rest[2:]
    else:
      pacc = rest[0]
      rest = rest[1:]
    (sidx, didx, rows_v, ones_v, h_sh, acc_sh, deg_sh,
     sem_g, sem_i, ss0, ss1) = rest
    sem_s = (ss0, ss1)

    cid = lax.axis_index("c")
    sid = lax.axis_index("s")
    tb = pl.multiple_of(sid * cpt, 8)
    r0 = pl.multiple_of(sid * ROWS_PER_TILE, 8)
    d0 = pl.multiple_of(sid * DEG_PER_TILE, 8)

    def gather_copies(slot, rowbase, grp, make):
      mk = pltpu.make_async_copy if make else pltpu.async_copy
      return [mk(h_sh.at[sidx.at[slot, rowbase + b]], rows_v.at[grp * RND + b],
                 sem_g) for b in range(RND)]

    def issue_gathers(slot, rowbase, grp):
      gather_copies(slot, rowbase, grp, False)

    def drain_gathers(slot, rowbase, grp):
      for c in gather_copies(slot, rowbase, grp, True):
        c.wait()

    def scatter_copies(slot, rowbase, grp, make):
      out = []
      for b in range(RND):
        args_r = (rows_v.at[grp * RND + b],
                  acc_sh.at[didx.at[slot, rowbase + b]], sem_s[grp])
        args_d = (ones_v, deg_sh.at[didx.at[slot, rowbase + b]], sem_s[grp])
        if make:
          out.append(pltpu.make_async_copy(*args_r))
          if with_deg:
            out.append(pltpu.make_async_copy(*args_d))
        else:
          out.append(pltpu.async_copy(*args_r, add=True))
          if with_deg:
            out.append(pltpu.async_copy(*args_d, add=True))
      return out

    def issue_scatters(slot, rowbase, grp):
      scatter_copies(slot, rowbase, grp, False)

    def drain_scatters(slot, rowbase, grp):
      for c in scatter_copies(slot, rowbase, grp, True):
        c.wait()

    def prefetch_idx(s, slot):
      off = pl.multiple_of(tb + s * SUP, 8)
      pltpu.async_copy(src2.at[pl.ds(off, SUP), :], sidx.at[slot], sem_i)
      pltpu.async_copy(dst2.at[pl.ds(off, SUP), :], didx.at[slot], sem_i)

    def wait_idx(slot):
      pltpu.make_async_copy(src2.at[pl.ds(tb, SUP), :], sidx.at[slot],
                            sem_i).wait()
      pltpu.make_async_copy(dst2.at[pl.ds(tb, SUP), :], didx.at[slot],
                            sem_i).wait()

    pltpu.sync_copy(h_pad.at[cid, pl.ds(r0, ROWS_PER_TILE), :],
                    h_sh.at[pl.ds(r0, ROWS_PER_TILE), :])
    pltpu.sync_copy(zrows.at[pl.ds(r0, ROWS_PER_TILE), :],
                    acc_sh.at[pl.ds(r0, ROWS_PER_TILE), :])
    if with_deg:
      pltpu.sync_copy(zflat.at[pl.ds(d0, DEG_PER_TILE)],
                      deg_sh.at[pl.ds(d0, DEG_PER_TILE)])
    pltpu.sync_copy(ones_in, ones_v)
    pltpu.sync_copy(src2.at[pl.ds(tb, SUP), :], sidx.at[0])
    pltpu.sync_copy(dst2.at[pl.ds(tb, SUP), :], didx.at[0])
    plsc.subcore_barrier()
    issue_gathers(0, 0, 0)

    nr = 2 * SUP // RND
    rps = SUP // RND
    pos = [((k * RND) // SUP, (k * RND) % SUP, k % 2) for k in range(nr)]

    def body(sp, carry):
      for k in range(nr):
        slot, rowbase, grp = pos[k]
        drain_gathers(slot, rowbase, grp)
        issue_scatters(slot, rowbase, grp)
        pslot, prowbase, pgrp = pos[k - 1]
        if k == 0:
          @pl.when(sp > 0)
          def _():
            drain_scatters(pslot, prowbase, pgrp)
          prefetch_idx(2 * sp + 1, 1)
        else:
          drain_scatters(pslot, prowbase, pgrp)
        if k == rps:
          @pl.when(sp < nsp - 1)
          def _():
            prefetch_idx(2 * sp + 2, 0)
        if k + 1 < nr:
          nslot, nrowbase, ngrp = pos[k + 1]
          if nslot != slot:
            wait_idx(nslot)
          issue_gathers(nslot, nrowbase, ngrp)
        else:
          @pl.when(sp < nsp - 1)
          def _():
            wait_idx(0)
            issue_gathers(0, 0, 0)
      return carry

    lax.fori_loop(0, nsp, body, 0)
    drain_scatters(*pos[nr - 1])
    plsc.subcore_barrier()

    pltpu.sync_copy(acc_sh.at[pl.ds(r0, ROWS_PER_TILE), :],
                    pacc.at[cid, pl.ds(r0, ROWS_PER_TILE), :])
    if with_deg:
      pltpu.sync_copy(deg_sh.at[pl.ds(d0, DEG_PER_TILE)],
                      pdeg.at[cid, pl.ds(d0, DEG_PER_TILE)])

  return agg


def _layer_body(h_ref, p_ref, deg_ref, ws_ref, wn_ref, b_ref, o_ref):
  agg = jnp.concatenate([p_ref[0], p_ref[1]], axis=1)
  agg = agg / jnp.maximum(deg_ref[0], 1.0)
  hs = lax.dot_general(h_ref[...], ws_ref[...], (((1,), (1,)), ((), ())),
                       preferred_element_type=jnp.float32)
  hn = lax.dot_general(agg, wn_ref[...], (((1,), (1,)), ((), ())),
                       preferred_element_type=jnp.float32)
  o_ref[...] = jnp.maximum(hs + hn + b_ref[...], 0.0)


def _layer_tc(h, pacc, deg3, ws, wn, b):
  nblk = N // ROW_BLK
  return pl.pallas_call(
      _layer_body,
      grid=(nblk,),
      in_specs=[
          pl.BlockSpec((ROW_BLK, D), lambda i: (i, 0)),
          pl.BlockSpec((NC, ROW_BLK, HD), lambda i: (0, i, 0)),
          pl.BlockSpec((1, ROW_BLK, 1), lambda i: (0, i, 0)),
          pl.BlockSpec((D, D), lambda i: (0, 0)),
          pl.BlockSpec((D, D), lambda i: (0, 0)),
          pl.BlockSpec((1, D), lambda i: (0, 0)),
      ],
      out_specs=pl.BlockSpec((ROW_BLK, D), lambda i: (i, 0)),
      out_shape=jax.ShapeDtypeStruct((N, D), jnp.float32),
  )(h, pacc, deg3, ws, wn, b)


def _pool_body(h_ref, bt_ref, w1_ref, b1_ref, w2_ref, o_ref,
               gsum, cnt):
  i = pl.program_id(0)

  @pl.when(i == 0)
  def _():
    gsum[...] = jnp.zeros_like(gsum)
    cnt[...] = jnp.zeros_like(cnt)

  oh = (bt_ref[...] == lax.broadcasted_iota(jnp.int32, (ROW_BLK, G), 1))
  oh = oh.astype(jnp.float32)
  gsum[...] += lax.dot_general(oh, h_ref[...], (((0,), (0,)), ((), ())),
                               preferred_element_type=jnp.float32)
  cnt[...] += jnp.sum(oh, axis=0)[:, None]

  @pl.when(i == pl.num_programs(0) - 1)
  def _():
    g = gsum[...] / jnp.maximum(cnt[...], 1.0)
    hh = lax.dot_general(g, w1_ref[...], (((1,), (1,)), ((), ())),
                         preferred_element_type=jnp.float32) + b1_ref[...]
    hh = jnp.maximum(hh, 0.0)
    o_ref[...] = jnp.sum(hh * w2_ref[...], axis=1, keepdims=True)


def _pool_tc(h, batch2, wh1, bh1, wh2):
  nblk = N // ROW_BLK
  return pl.pallas_call(
      _pool_body,
      grid=(nblk,),
      in_specs=[
          pl.BlockSpec((ROW_BLK, D), lambda i: (i, 0)),
          pl.BlockSpec((ROW_BLK, 1), lambda i: (i, 0)),
          pl.BlockSpec((D, D), lambda i: (0, 0)),
          pl.BlockSpec((1, D), lambda i: (0, 0)),
          pl.BlockSpec((1, D), lambda i: (0, 0)),
      ],
      out_specs=pl.BlockSpec((G, 1), lambda i: (0, 0)),
      out_shape=jax.ShapeDtypeStruct((G, 1), jnp.float32),
      scratch_shapes=[
          pltpu.VMEM((G, D), jnp.float32),
          pltpu.VMEM((G, 1), jnp.float32),
      ],
  )(h, batch2, wh1, bh1, wh2)


def kernel(x, edge_index, batch, Ws0, bs0, Wn0, bn0, Ws1, bs1, Wn1, bn1,
           Ws2, bs2, Wn2, bn2, Wh1, bh1, Wh2, bh2):
  e = edge_index.shape[1]
  cpt = -(-e // (NS * CHUNK))
  cpt = -(-cpt // (2 * SUP)) * (2 * SUP)
  epad = NS * cpt * CHUNK
  pad = epad - e
  src = jnp.concatenate([edge_index[0],
                         jnp.zeros((pad,), jnp.int32)]).reshape(-1, CHUNK)
  dst = jnp.concatenate([edge_index[1],
                         jnp.full((pad,), N, jnp.int32)]).reshape(-1, CHUNK)
  rowpad = jnp.zeros((NC, NP - N, HD), jnp.float32)
  zrows = jnp.zeros((NP, HD), jnp.float32)
  zflat = jnp.zeros((NDP,), jnp.float32)
  ones_in = jnp.ones((CHUNK,), jnp.float32)
  agg_first = _make_agg(cpt, True)
  agg_rest = _make_agg(cpt, False)

  h = x
  pdeg = None
  for li, (ws, bs, wn, bn) in enumerate(
      ((Ws0, bs0, Wn0, bn0), (Ws1, bs1, Wn1, bn1), (Ws2, bs2, Wn2, bn2))):
    h_pad = jnp.concatenate(
        [h.reshape(N, NC, HD).transpose(1, 0, 2), rowpad], axis=1)
    if li == 0:
      pacc, pdeg = agg_first(h_pad, src, dst, zrows, zflat, ones_in)
    else:
      (pacc,) = agg_rest(h_pad, src, dst, zrows, zflat, ones_in)
    h = _layer_tc(h, pacc, pdeg.reshape(NC, NDP, 1), ws, wn,
                  (bs + bn).reshape(1, D))

  out = _pool_tc(h, batch.reshape(N, 1), Wh1, bh1.reshape(1, D), Wh2)
  return out.reshape(-1) + bh2
